# baseline (speedup 1.0000x reference)
import functools
import jax
import jax.numpy as jnp
import numpy as np
from jax import lax
from jax.experimental import pallas as pl
from jax.experimental.pallas import tpu as pltpu
from jax.experimental.pallas import tpu_sc as plsc

NC = 2
NS = 16
LANES = 16
C = 128
NBUF = 8


def _mesh():
    return plsc.VectorSubcoreMesh(
        core_axis_name="c", subcore_axis_name="s", num_cores=NC, num_subcores=NS
    )


def _rsqrt_newton(x):
    i = plsc.bitcast(x, jnp.int32)
    i = 0x5F3759DF - (i >> 1)
    y = plsc.bitcast(i, jnp.float32)
    for _ in range(3):
        y = y * (1.5 - 0.5 * x * y * y)
    return y


def _deg_kernel(n_nodes, chunks_per_worker, ei_hbm, w_hbm, dinv4_hbm,
                dst_v, w_v, ssems, degb, dvb, acc_sh, zbuf):
    cid = lax.axis_index("c")
    sid = lax.axis_index("s")
    wid = sid * NC + cid
    cpw = 2 * chunks_per_worker
    start = sid * cpw
    n_pad = acc_sh.shape[0]

    @pl.when(sid == 0)
    def _():
        def zero_body(i, _):
            zbuf[pl.ds(i * LANES, LANES)] = jnp.zeros((LANES,), jnp.float32)
            return _
        lax.fori_loop(0, n_pad // LANES, zero_body, None)
        pltpu.sync_copy(zbuf, acc_sh)

    plsc.subcore_barrier()

    pltpu.sync_copy(ei_hbm.at[1, pl.ds(start, cpw)], dst_v)
    pltpu.sync_copy(w_hbm.at[pl.ds(start * C, cpw * C)], w_v)

    def ring_body(jo, _):
        for b in range(NBUF):
            j = jo * NBUF + b

            @pl.when(j >= NBUF)
            def _():
                pltpu.make_async_copy(
                    w_v.at[pl.ds((j - NBUF) * C, C)],
                    acc_sh.at[dst_v.at[j - NBUF]], ssems[b]
                ).wait()

            pltpu.async_copy(w_v.at[pl.ds(j * C, C)],
                             acc_sh.at[dst_v.at[j]], ssems[b], add=True)
        return _
    lax.fori_loop(0, cpw // NBUF, ring_body, None)

    for j in range(cpw - NBUF, cpw):
        pltpu.make_async_copy(
            w_v.at[pl.ds(j * C, C)], acc_sh.at[dst_v.at[j]], ssems[j % NBUF]
        ).wait()

    plsc.subcore_barrier()

    rows_per = 100
    nodes_per = rows_per * 4

    @pl.when(wid < n_nodes // nodes_per)
    def _():
        nbase = wid * nodes_per
        pltpu.sync_copy(acc_sh.at[pl.ds(nbase, nodes_per)], degb)

        def dinv_body(g, _):
            dv = _rsqrt_newton(degb[pl.ds(g * LANES, LANES)] + 1.0)
            for t in range(LANES):
                n = g * LANES + t
                splat = jnp.full((LANES,), dv[t])
                dvb[n // 4, pl.ds(32 * (n % 4), LANES)] = splat
                dvb[n // 4, pl.ds(32 * (n % 4) + LANES, LANES)] = splat
            return _
        lax.fori_loop(0, nodes_per // LANES, dinv_body, None)
        pltpu.sync_copy(dvb, dinv4_hbm.at[pl.ds(wid * rows_per, rows_per)])


def _agg_kernel(n_nodes, chunks_per_worker, ei_hbm, w_hbm, y_hbm,
                out_hbm, src_v, dst_v, w_v, rows, gsems, ssems, acc_sh, y_sh):
    cid = lax.axis_index("c")
    sid = lax.axis_index("s")
    wid = sid * NC + cid
    start = wid * chunks_per_worker
    cpw = chunks_per_worker

    @pl.when(sid == 0)
    def _():
        pltpu.sync_copy(y_hbm, acc_sh)

    plsc.subcore_barrier()

    pltpu.sync_copy(ei_hbm.at[0, pl.ds(start, cpw)], src_v)
    pltpu.sync_copy(ei_hbm.at[1, pl.ds(start, cpw)], dst_v)
    pltpu.sync_copy(w_hbm.at[pl.ds(start * C, cpw * C)], w_v)

    def start_gather(j, b):
        pltpu.async_copy(y_hbm.at[src_v.at[j]], rows[b], gsems[b])

    def wait_gather(j, b):
        pltpu.make_async_copy(y_hbm.at[src_v.at[j]], rows[b], gsems[b]).wait()

    def start_scatter(j, b):
        pltpu.async_copy(rows[b], acc_sh.at[dst_v.at[j]], ssems[b], add=True)

    def wait_scatter(j, b):
        pltpu.make_async_copy(rows[b], acc_sh.at[dst_v.at[j]],
                              ssems[b]).wait()

    def scale(j, b):
        base = j * C
        buf = rows[b]

        def scale_body(g, _):
            gbase = g * LANES
            wv = w_v[pl.ds(base + gbase, LANES)]
            for t in range(LANES):
                k = gbase + t
                wk = jnp.full((LANES,), wv[t])
                buf[k, pl.ds(0, LANES)] = buf[k, pl.ds(0, LANES)] * wk
                buf[k, pl.ds(LANES, LANES)] = buf[k, pl.ds(LANES, LANES)] * wk
            return _
        lax.fori_loop(0, C // LANES, scale_body, None)

    start_gather(0, 0)

    def ring_body(jo, _):
        for b in range(NBUF):
            j = jo * NBUF + b

            @pl.when(j >= NBUF)
            def _():
                wait_scatter(j - NBUF, b)

            @pl.when(j + 1 < cpw)
            def _():
                start_gather(j + 1, (b + 1) % NBUF)

            wait_gather(j, b)
            scale(j, b)
            start_scatter(j, b)
        return _
    lax.fori_loop(0, cpw // NBUF, ring_body, None)

    for j in range(cpw - NBUF, cpw):
        wait_scatter(j, j % NBUF)

    plsc.subcore_barrier()

    @pl.when(sid == 0)
    def _():
        pltpu.sync_copy(acc_sh, out_hbm.at[cid])


def _tc_k1(x4_ref, w1x_ref, dinv4_ref, y4_ref):
    xw4 = jnp.dot(x4_ref[...], w1x_ref[...], preferred_element_type=jnp.float32)
    y4_ref[...] = xw4 * dinv4_ref[...]


def _tc_k2(accp_ref, y4_ref, dinv4_ref, b14_ref, w2x_ref, y24_ref):
    dinv4 = dinv4_ref[...]
    t4 = accp_ref[0] + accp_ref[1] - y4_ref[...]
    h14 = jnp.maximum(dinv4 * t4 + b14_ref[...][None, :], 0.0)
    xw24 = jnp.dot(h14, w2x_ref[...], preferred_element_type=jnp.float32)
    y24_ref[...] = xw24 * dinv4


def _tc_k3(accp_ref, y24_ref, dinv4_ref, b24_ref, wlb_ref, bl_ref, out_ref):
    dinv4 = dinv4_ref[...]
    t4 = accp_ref[0] + accp_ref[1] - y24_ref[...]
    h24 = jnp.maximum(dinv4 * t4 + b24_ref[...][None, :], 0.0)
    res = jnp.dot(h24, wlb_ref[...], preferred_element_type=jnp.float32)
    out_ref[...] = res + bl_ref[0]


@jax.jit
def kernel(x, edge_index, edge_weight, W1, b1, W2, b2, Wl, bl):
    n_nodes, d_in = x.shape
    h = W1.shape[1]
    e = edge_index.shape[1]
    npack = n_nodes // 4

    n_workers = NC * NS
    chunks = -(-e // C)
    cpw = -(-chunks // n_workers)
    cpw = -(-cpw // 8) * 8
    e_pad = n_workers * cpw * C

    pad = e_pad - e
    w = edge_weight
    ei = edge_index
    if pad:
        fill = (jnp.arange(pad, dtype=jnp.int32) * 97) % n_nodes
        ei = jnp.concatenate([ei, jnp.stack([fill, fill])], axis=1)
        w = jnp.concatenate([w, jnp.zeros((pad,), jnp.float32)])
    ei3d = ei.reshape(2, -1, C)

    x4 = x.reshape(npack, 4 * d_in)
    W1x = jnp.zeros((4 * d_in, 4 * h), W1.dtype)
    for b in range(4):
        W1x = W1x.at[b * d_in:(b + 1) * d_in, b * h:(b + 1) * h].set(W1)
    W2x = jnp.zeros((4 * h, 4 * h), W2.dtype)
    for b in range(4):
        W2x = W2x.at[b * h:(b + 1) * h, b * h:(b + 1) * h].set(W2)
    WlB = jnp.zeros((4 * h, 128), Wl.dtype)
    for b in range(4):
        WlB = WlB.at[b * h:(b + 1) * h, b * 32].set(Wl[:, 0])
    b14 = jnp.tile(b1, 4)
    b24 = jnp.tile(b2, 4)

    mesh = _mesh()
    n_pad_nodes = -(-n_nodes // 128) * 128

    deg_fn = pl.kernel(
        functools.partial(_deg_kernel, n_nodes, cpw),
        out_type=jax.ShapeDtypeStruct((npack, 128), jnp.float32),
        mesh=mesh,
        compiler_params=pltpu.CompilerParams(
            use_tc_tiling_on_sc=False, needs_layout_passes=False),
        scratch_types=[
            pltpu.MemorySpace.VMEM((2 * cpw, C), jnp.int32),
            pltpu.MemorySpace.VMEM((2 * cpw * C,), jnp.float32),
            tuple(pltpu.SemaphoreType.DMA for _ in range(NBUF)),
            pltpu.MemorySpace.VMEM((400,), jnp.float32),
            pltpu.MemorySpace.VMEM((100, 128), jnp.float32),
            pltpu.MemorySpace.VMEM_SHARED((n_pad_nodes,), jnp.float32),
            pltpu.MemorySpace.VMEM((n_pad_nodes,), jnp.float32),
        ],
    )
    dinv4 = deg_fn(ei3d, w)

    agg_fn = pl.kernel(
        functools.partial(_agg_kernel, n_nodes, cpw),
        out_type=jax.ShapeDtypeStruct((NC, n_nodes, h), jnp.float32),
        mesh=mesh,
        compiler_params=pltpu.CompilerParams(use_tc_tiling_on_sc=False),
        scratch_types=[
            pltpu.MemorySpace.VMEM((cpw, C), jnp.int32),
            pltpu.MemorySpace.VMEM((cpw, C), jnp.int32),
            pltpu.MemorySpace.VMEM((cpw * C,), jnp.float32),
            tuple(pltpu.MemorySpace.VMEM((C, h), jnp.float32)
                  for _ in range(NBUF)),
            tuple(pltpu.SemaphoreType.DMA for _ in range(NBUF)),
            tuple(pltpu.SemaphoreType.DMA for _ in range(NBUF)),
            pltpu.MemorySpace.VMEM_SHARED((n_nodes, h), jnp.float32),
            pltpu.MemorySpace.VMEM_SHARED((n_nodes, h), jnp.float32),
        ],
    )

    y4 = pl.pallas_call(
        _tc_k1,
        out_shape=jax.ShapeDtypeStruct((npack, 128), jnp.float32),
    )(x4, W1x, dinv4)

    accp1 = agg_fn(ei3d, w, y4.reshape(n_nodes, h))

    y24 = pl.pallas_call(
        _tc_k2,
        out_shape=jax.ShapeDtypeStruct((npack, 128), jnp.float32),
    )(accp1.reshape(NC, npack, 128), y4, dinv4, b14, W2x)

    accp2 = agg_fn(ei3d, w, y24.reshape(n_nodes, h))

    out4 = pl.pallas_call(
        _tc_k3,
        out_shape=jax.ShapeDtypeStruct((npack, 128), jnp.float32),
    )(accp2.reshape(NC, npack, 128), y24, dinv4, b24, WlB, bl)

    return out4[:, ::32].reshape(-1)


# speedup vs baseline: 1.1252x; 1.1252x over previous
import functools
import jax
import jax.numpy as jnp
import numpy as np
from jax import lax
from jax.experimental import pallas as pl
from jax.experimental.pallas import tpu as pltpu
from jax.experimental.pallas import tpu_sc as plsc

NC = 2
NS = 16
LANES = 16
C = 128
NBUF = 8


def _mesh():
    return plsc.VectorSubcoreMesh(
        core_axis_name="c", subcore_axis_name="s", num_cores=NC, num_subcores=NS
    )


def _rsqrt_newton(x):
    i = plsc.bitcast(x, jnp.int32)
    i = 0x5F3759DF - (i >> 1)
    y = plsc.bitcast(i, jnp.float32)
    for _ in range(3):
        y = y * (1.5 - 0.5 * x * y * y)
    return y


def _deg_kernel(n_nodes, chunks_per_worker, ei_hbm, w_hbm, dinv4_hbm,
                dst_v, w_v, ssems, degb, dvb, acc_sh, zbuf):
    cid = lax.axis_index("c")
    sid = lax.axis_index("s")
    wid = sid * NC + cid
    cpw = 2 * chunks_per_worker
    start = sid * cpw
    n_pad = acc_sh.shape[0]

    @pl.when(sid == 0)
    def _():
        def zero_body(i, _):
            zbuf[pl.ds(i * LANES, LANES)] = jnp.zeros((LANES,), jnp.float32)
            return _
        lax.fori_loop(0, n_pad // LANES, zero_body, None)
        pltpu.sync_copy(zbuf, acc_sh)

    plsc.subcore_barrier()

    pltpu.sync_copy(ei_hbm.at[1, pl.ds(start, cpw)], dst_v)
    pltpu.sync_copy(w_hbm.at[pl.ds(start * C, cpw * C)], w_v)

    def ring_body(jo, _):
        for b in range(NBUF):
            j = jo * NBUF + b

            @pl.when(j >= NBUF)
            def _():
                pltpu.make_async_copy(
                    w_v.at[pl.ds((j - NBUF) * C, C)],
                    acc_sh.at[dst_v.at[j - NBUF]], ssems[b]
                ).wait()

            pltpu.async_copy(w_v.at[pl.ds(j * C, C)],
                             acc_sh.at[dst_v.at[j]], ssems[b], add=True)
        return _
    lax.fori_loop(0, cpw // NBUF, ring_body, None)

    for j in range(cpw - NBUF, cpw):
        pltpu.make_async_copy(
            w_v.at[pl.ds(j * C, C)], acc_sh.at[dst_v.at[j]], ssems[j % NBUF]
        ).wait()

    plsc.subcore_barrier()

    rows_per = 100
    nodes_per = rows_per * 4

    @pl.when(wid < n_nodes // nodes_per)
    def _():
        nbase = wid * nodes_per
        pltpu.sync_copy(acc_sh.at[pl.ds(nbase, nodes_per)], degb)

        def dinv_body(g, _):
            dv = _rsqrt_newton(degb[pl.ds(g * LANES, LANES)] + 1.0)
            for t in range(LANES):
                n = g * LANES + t
                splat = jnp.full((LANES,), dv[t])
                dvb[n // 4, pl.ds(32 * (n % 4), LANES)] = splat
                dvb[n // 4, pl.ds(32 * (n % 4) + LANES, LANES)] = splat
            return _
        lax.fori_loop(0, nodes_per // LANES, dinv_body, None)
        pltpu.sync_copy(dvb, dinv4_hbm.at[pl.ds(wid * rows_per, rows_per)])


def _agg_kernel(n_nodes, chunks_per_worker, ei_hbm, w_hbm, y_hbm,
                out_hbm, src_v, dst_v, w_v, rows, gsems, ssems, acc_sh, y_sh):
    cid = lax.axis_index("c")
    sid = lax.axis_index("s")
    wid = sid * NC + cid
    start = wid * chunks_per_worker
    cpw = chunks_per_worker

    @pl.when(sid == 0)
    def _():
        pltpu.sync_copy(y_hbm, acc_sh)

    @pl.when(sid == 1)
    def _():
        pltpu.sync_copy(y_hbm, y_sh)

    plsc.subcore_barrier()

    pltpu.sync_copy(ei_hbm.at[0, pl.ds(start, cpw)], src_v)
    pltpu.sync_copy(ei_hbm.at[1, pl.ds(start, cpw)], dst_v)
    pltpu.sync_copy(w_hbm.at[pl.ds(start * C, cpw * C)], w_v)

    def start_gather(j, b):
        pltpu.async_copy(y_sh.at[src_v.at[j]], rows[b], gsems[b])

    def wait_gather(j, b):
        pltpu.make_async_copy(y_sh.at[src_v.at[j]], rows[b], gsems[b]).wait()

    def start_scatter(j, b):
        pltpu.async_copy(rows[b], acc_sh.at[dst_v.at[j]], ssems[b], add=True)

    def wait_scatter(j, b):
        pltpu.make_async_copy(rows[b], acc_sh.at[dst_v.at[j]],
                              ssems[b]).wait()

    def scale(j, b):
        base = j * C
        buf = rows[b]

        def scale_body(g, _):
            gbase = g * LANES
            wv = w_v[pl.ds(base + gbase, LANES)]
            for t in range(LANES):
                k = gbase + t
                wk = jnp.full((LANES,), wv[t])
                buf[k, pl.ds(0, LANES)] = buf[k, pl.ds(0, LANES)] * wk
                buf[k, pl.ds(LANES, LANES)] = buf[k, pl.ds(LANES, LANES)] * wk
            return _
        lax.fori_loop(0, C // LANES, scale_body, None)

    start_gather(0, 0)

    def ring_body(jo, _):
        for b in range(NBUF):
            j = jo * NBUF + b

            @pl.when(j >= NBUF)
            def _():
                wait_scatter(j - NBUF, b)

            @pl.when(j + 1 < cpw)
            def _():
                start_gather(j + 1, (b + 1) % NBUF)

            wait_gather(j, b)
            scale(j, b)
            start_scatter(j, b)
        return _
    lax.fori_loop(0, cpw // NBUF, ring_body, None)

    for j in range(cpw - NBUF, cpw):
        wait_scatter(j, j % NBUF)

    plsc.subcore_barrier()

    @pl.when(sid == 0)
    def _():
        pltpu.sync_copy(acc_sh, out_hbm.at[cid])


def _tc_k1(x4_ref, w1x_ref, dinv4_ref, y4_ref):
    xw4 = jnp.dot(x4_ref[...], w1x_ref[...], preferred_element_type=jnp.float32)
    y4_ref[...] = xw4 * dinv4_ref[...]


def _tc_k2(accp_ref, y4_ref, dinv4_ref, b14_ref, w2x_ref, y24_ref):
    dinv4 = dinv4_ref[...]
    t4 = accp_ref[0] + accp_ref[1] - y4_ref[...]
    h14 = jnp.maximum(dinv4 * t4 + b14_ref[...][None, :], 0.0)
    xw24 = jnp.dot(h14, w2x_ref[...], preferred_element_type=jnp.float32)
    y24_ref[...] = xw24 * dinv4


def _tc_k3(accp_ref, y24_ref, dinv4_ref, b24_ref, wlb_ref, bl_ref, out_ref):
    dinv4 = dinv4_ref[...]
    t4 = accp_ref[0] + accp_ref[1] - y24_ref[...]
    h24 = jnp.maximum(dinv4 * t4 + b24_ref[...][None, :], 0.0)
    res = jnp.dot(h24, wlb_ref[...], preferred_element_type=jnp.float32)
    out_ref[...] = res + bl_ref[0]


@jax.jit
def kernel(x, edge_index, edge_weight, W1, b1, W2, b2, Wl, bl):
    n_nodes, d_in = x.shape
    h = W1.shape[1]
    e = edge_index.shape[1]
    npack = n_nodes // 4

    n_workers = NC * NS
    chunks = -(-e // C)
    cpw = -(-chunks // n_workers)
    cpw = -(-cpw // 8) * 8
    e_pad = n_workers * cpw * C

    pad = e_pad - e
    w = edge_weight
    ei = edge_index
    if pad:
        fill = (jnp.arange(pad, dtype=jnp.int32) * 97) % n_nodes
        ei = jnp.concatenate([ei, jnp.stack([fill, fill])], axis=1)
        w = jnp.concatenate([w, jnp.zeros((pad,), jnp.float32)])
    ei3d = ei.reshape(2, -1, C)

    x4 = x.reshape(npack, 4 * d_in)
    W1x = jnp.zeros((4 * d_in, 4 * h), W1.dtype)
    for b in range(4):
        W1x = W1x.at[b * d_in:(b + 1) * d_in, b * h:(b + 1) * h].set(W1)
    W2x = jnp.zeros((4 * h, 4 * h), W2.dtype)
    for b in range(4):
        W2x = W2x.at[b * h:(b + 1) * h, b * h:(b + 1) * h].set(W2)
    WlB = jnp.zeros((4 * h, 128), Wl.dtype)
    for b in range(4):
        WlB = WlB.at[b * h:(b + 1) * h, b].set(Wl[:, 0])
    b14 = jnp.tile(b1, 4)
    b24 = jnp.tile(b2, 4)

    mesh = _mesh()
    n_pad_nodes = -(-n_nodes // 128) * 128

    deg_fn = pl.kernel(
        functools.partial(_deg_kernel, n_nodes, cpw),
        out_type=jax.ShapeDtypeStruct((npack, 128), jnp.float32),
        mesh=mesh,
        compiler_params=pltpu.CompilerParams(
            use_tc_tiling_on_sc=False, needs_layout_passes=False),
        scratch_types=[
            pltpu.MemorySpace.VMEM((2 * cpw, C), jnp.int32),
            pltpu.MemorySpace.VMEM((2 * cpw * C,), jnp.float32),
            tuple(pltpu.SemaphoreType.DMA for _ in range(NBUF)),
            pltpu.MemorySpace.VMEM((400,), jnp.float32),
            pltpu.MemorySpace.VMEM((100, 128), jnp.float32),
            pltpu.MemorySpace.VMEM_SHARED((n_pad_nodes,), jnp.float32),
            pltpu.MemorySpace.VMEM((n_pad_nodes,), jnp.float32),
        ],
    )
    dinv4 = deg_fn(ei3d, w)

    agg_fn = pl.kernel(
        functools.partial(_agg_kernel, n_nodes, cpw),
        out_type=jax.ShapeDtypeStruct((NC, n_nodes, h), jnp.float32),
        mesh=mesh,
        compiler_params=pltpu.CompilerParams(use_tc_tiling_on_sc=False),
        scratch_types=[
            pltpu.MemorySpace.VMEM((cpw, C), jnp.int32),
            pltpu.MemorySpace.VMEM((cpw, C), jnp.int32),
            pltpu.MemorySpace.VMEM((cpw * C,), jnp.float32),
            tuple(pltpu.MemorySpace.VMEM((C, h), jnp.float32)
                  for _ in range(NBUF)),
            tuple(pltpu.SemaphoreType.DMA for _ in range(NBUF)),
            tuple(pltpu.SemaphoreType.DMA for _ in range(NBUF)),
            pltpu.MemorySpace.VMEM_SHARED((n_nodes, h), jnp.float32),
            pltpu.MemorySpace.VMEM_SHARED((n_nodes, h), jnp.float32),
        ],
    )

    y4 = pl.pallas_call(
        _tc_k1,
        out_shape=jax.ShapeDtypeStruct((npack, 128), jnp.float32),
    )(x4, W1x, dinv4)

    accp1 = agg_fn(ei3d, w, y4.reshape(n_nodes, h))

    y24 = pl.pallas_call(
        _tc_k2,
        out_shape=jax.ShapeDtypeStruct((npack, 128), jnp.float32),
    )(accp1.reshape(NC, npack, 128), y4, dinv4, b14, W2x)

    accp2 = agg_fn(ei3d, w, y24.reshape(n_nodes, h))

    out4 = pl.pallas_call(
        _tc_k3,
        out_shape=jax.ShapeDtypeStruct((npack, 128), jnp.float32),
    )(accp2.reshape(NC, npack, 128), y24, dinv4, b24, WlB, bl)

    return out4[:, :4].reshape(-1)
